# Initial kernel scaffold; baseline (speedup 1.0000x reference)
import jax
import jax.numpy as jnp
from jax.experimental import pallas as pl


def kernel(x_emb, x_cont, emb0, emb1, emb2, emb3, emb4, emb5, emb6, W0, b0, g0, be0, rm0, rv0, W1, b1, g1, be1, rm1, rv1, W2, b2, g2, be2, rm2, rv2, W3, b3):
    raise NotImplementedError("write your pallas kernel here")


# speedup vs baseline: 1.4510x; 1.4510x over previous
import functools

import jax
import jax.numpy as jnp
from jax import lax
from jax.experimental import pallas as pl
from jax.experimental.pallas import tpu as pltpu
from jax.experimental.pallas import tpu_sc as plsc

BATCH = 16384
L = 16
EMB_ROWS = (1559, 16, 5, 3, 4, 10, 4)
EMB_DIMS = (50, 8, 3, 2, 3, 5, 3)
NCH = tuple(-(-d // L) for d in EMB_DIMS)
N_CONT = 15
CHUNKS = sum(NCH) + 1
XW = CHUNKS * L
IDX_MINOR = 128


def _sc_gather(src, cidx):
    info = plsc.get_sparse_core_info()
    nw = info.num_cores * info.num_subcores
    nidx = BATCH * CHUNKS
    per_w = nidx // nw
    n_chunk = per_w // IDX_MINOR
    mesh = plsc.VectorSubcoreMesh(core_axis_name="c", subcore_axis_name="s")

    @functools.partial(
        pl.kernel,
        mesh=mesh,
        compiler_params=pltpu.CompilerParams(use_tc_tiling_on_sc=False),
        out_type=jax.ShapeDtypeStruct((nidx, L), jnp.float32),
        scratch_types=[
            pltpu.VMEM((n_chunk, IDX_MINOR), jnp.int32),
            pltpu.VMEM((per_w, L), jnp.float32),
            pltpu.SemaphoreType.DMA,
        ],
    )
    def k(src_hbm, cidx_hbm, out_hbm, idx_v, rows_v, sem):
        wid = lax.axis_index("s") * info.num_cores + lax.axis_index("c")
        pltpu.sync_copy(cidx_hbm.at[wid], idx_v)

        def body(j, carry):
            pltpu.async_copy(
                src_hbm.at[idx_v.at[j]],
                rows_v.at[pl.ds(j * IDX_MINOR, IDX_MINOR)],
                sem,
            ).wait()
            return carry

        lax.fori_loop(0, n_chunk, body, 0)
        pltpu.sync_copy(rows_v, out_hbm.at[pl.ds(wid * per_w, per_w)])

    return k(src, cidx)


def _mlp_body(x_ref, w0_ref, b0_ref, g0_ref, be0_ref, rm0_ref, rv0_ref,
              w1_ref, b1_ref, g1_ref, be1_ref, rm1_ref, rv1_ref,
              w2_ref, b2_ref, g2_ref, be2_ref, rm2_ref, rv2_ref,
              w3_ref, b3_ref, out_ref):
    h = x_ref[...]
    for w_ref, b_ref, g_ref, be_ref, rm_ref, rv_ref in (
        (w0_ref, b0_ref, g0_ref, be0_ref, rm0_ref, rv0_ref),
        (w1_ref, b1_ref, g1_ref, be1_ref, rm1_ref, rv1_ref),
        (w2_ref, b2_ref, g2_ref, be2_ref, rm2_ref, rv2_ref),
    ):
        y = lax.dot_general(h, w_ref[...], (((1,), (1,)), ((), ())),
                            preferred_element_type=jnp.float32)
        y = y + b_ref[...][None, :]
        scale = (g_ref[...] * lax.rsqrt(rv_ref[...] + 1e-5))[None, :]
        y = (y - rm_ref[...][None, :]) * scale + be_ref[...][None, :]
        h = jnp.maximum(y, 0.0)
    o = lax.dot_general(h, w3_ref[...], (((1,), (1,)), ((), ())),
                        preferred_element_type=jnp.float32)
    out_ref[...] = o + b3_ref[0]


def _mlp(x, w0p, b0, g0, be0, rm0, rv0, w1, b1, g1, be1, rm1, rv1,
         w2, b2, g2, be2, rm2, rv2, w3, b3, block_b=512):
    grid = (BATCH // block_b,)

    def full2(shape):
        return pl.BlockSpec(shape, lambda i: (0, 0))

    def full1(shape):
        return pl.BlockSpec(shape, lambda i: (0,))

    h0, h1, h2 = w0p.shape[0], w1.shape[0], w2.shape[0]
    in_specs = [
        pl.BlockSpec((block_b, XW), lambda i: (i, 0)),
        full2(w0p.shape), full1((h0,)), full1((h0,)), full1((h0,)),
        full1((h0,)), full1((h0,)),
        full2(w1.shape), full1((h1,)), full1((h1,)), full1((h1,)),
        full1((h1,)), full1((h1,)),
        full2(w2.shape), full1((h2,)), full1((h2,)), full1((h2,)),
        full1((h2,)), full1((h2,)),
        full2(w3.shape),
        pl.BlockSpec(memory_space=pltpu.MemorySpace.SMEM),
    ]
    return pl.pallas_call(
        _mlp_body,
        grid=grid,
        in_specs=in_specs,
        out_specs=pl.BlockSpec((block_b, 8), lambda i: (i, 0)),
        out_shape=jax.ShapeDtypeStruct((BATCH, 8), jnp.float32),
    )(x, w0p, b0, g0, be0, rm0, rv0, w1, b1, g1, be1, rm1, rv1,
      w2, b2, g2, be2, rm2, rv2, w3, b3)


def _pack_source(tables, x_cont):
    parts = []
    for t, nch in zip(tables, NCH):
        n, d = t.shape
        tp = jnp.pad(t, ((0, 0), (0, nch * L - d)))
        parts.append(tp.reshape(n * nch, L))
    parts.append(jnp.pad(x_cont, ((0, 0), (0, L - N_CONT))))
    return jnp.concatenate(parts, axis=0)


def _chunk_indices(x_emb):
    xi = x_emb.astype(jnp.int32)
    bases = []
    base = 0
    for n, nch in zip(EMB_ROWS, NCH):
        bases.append(base)
        base += n * nch
    cols = []
    for f in range(len(EMB_ROWS)):
        for j in range(NCH[f]):
            cols.append(bases[f] + xi[:, f] * NCH[f] + j)
    cols.append(base + jnp.arange(BATCH, dtype=jnp.int32))
    nw = plsc.get_sparse_core_info().num_cores * plsc.get_sparse_core_info().num_subcores
    return jnp.stack(cols, axis=1).reshape(nw, -1, IDX_MINOR)


def _pad_w0(w0):
    segs = []
    off = 0
    for d, nch in zip(EMB_DIMS, NCH):
        segs.append(jnp.pad(w0[:, off:off + d], ((0, 0), (0, nch * L - d))))
        off += d
    segs.append(jnp.pad(w0[:, off:off + N_CONT], ((0, 0), (0, L - N_CONT))))
    return jnp.concatenate(segs, axis=1)


def kernel(x_emb, x_cont, emb0, emb1, emb2, emb3, emb4, emb5, emb6,
           W0, b0, g0, be0, rm0, rv0,
           W1, b1, g1, be1, rm1, rv1,
           W2, b2, g2, be2, rm2, rv2,
           W3, b3):
    tables = (emb0, emb1, emb2, emb3, emb4, emb5, emb6)
    src = _pack_source(tables, x_cont)
    cidx = _chunk_indices(x_emb)
    x = _sc_gather(src, cidx).reshape(BATCH, XW)
    w0p = _pad_w0(W0)
    w3p = jnp.pad(W3, ((0, 7), (0, 0)))
    out = _mlp(x, w0p, b0, g0, be0, rm0, rv0, W1, b1, g1, be1, rm1, rv1,
               W2, b2, g2, be2, rm2, rv2, w3p, b3)
    return out[:, 0]


# speedup vs baseline: 1.4518x; 1.0006x over previous
import functools

import jax
import jax.numpy as jnp
from jax import lax
from jax.experimental import pallas as pl
from jax.experimental.pallas import tpu as pltpu
from jax.experimental.pallas import tpu_sc as plsc

BATCH = 16384
L = 16
EMB_ROWS = (1559, 16, 5, 3, 4, 10, 4)
EMB_DIMS = (50, 8, 3, 2, 3, 5, 3)
NCH = tuple(-(-d // L) for d in EMB_DIMS)
N_CONT = 15
CHUNKS = sum(NCH) + 1
XW = CHUNKS * L
IDX_MINOR = 128


def _sc_gather(src, cidx):
    info = plsc.get_sparse_core_info()
    nw = info.num_cores * info.num_subcores
    nidx = BATCH * CHUNKS
    per_w = nidx // nw
    n_chunk = per_w // IDX_MINOR
    mesh = plsc.VectorSubcoreMesh(core_axis_name="c", subcore_axis_name="s")

    @functools.partial(
        pl.kernel,
        mesh=mesh,
        compiler_params=pltpu.CompilerParams(use_tc_tiling_on_sc=False),
        out_type=jax.ShapeDtypeStruct((nidx, L), jnp.float32),
        scratch_types=[
            pltpu.VMEM((n_chunk, IDX_MINOR), jnp.int32),
            pltpu.VMEM((per_w, L), jnp.float32),
            pltpu.SemaphoreType.DMA,
        ],
    )
    def k(src_hbm, cidx_hbm, out_hbm, idx_v, rows_v, sem):
        wid = lax.axis_index("s") * info.num_cores + lax.axis_index("c")
        pltpu.sync_copy(cidx_hbm.at[wid], idx_v)

        def body(j, carry):
            pltpu.async_copy(
                src_hbm.at[idx_v.at[j]],
                rows_v.at[pl.ds(j * IDX_MINOR, IDX_MINOR)],
                sem,
            )
            return carry

        lax.fori_loop(0, n_chunk, body, 0)
        pltpu.make_async_copy(
            out_hbm.at[pl.ds(wid * per_w, per_w)], rows_v, sem
        ).wait()
        pltpu.sync_copy(rows_v, out_hbm.at[pl.ds(wid * per_w, per_w)])

    return k(src, cidx)


def _mlp_body(x_ref, w0_ref, b0_ref, g0_ref, be0_ref, rm0_ref, rv0_ref,
              w1_ref, b1_ref, g1_ref, be1_ref, rm1_ref, rv1_ref,
              w2_ref, b2_ref, g2_ref, be2_ref, rm2_ref, rv2_ref,
              w3_ref, b3_ref, out_ref):
    h = x_ref[...]
    for w_ref, b_ref, g_ref, be_ref, rm_ref, rv_ref in (
        (w0_ref, b0_ref, g0_ref, be0_ref, rm0_ref, rv0_ref),
        (w1_ref, b1_ref, g1_ref, be1_ref, rm1_ref, rv1_ref),
        (w2_ref, b2_ref, g2_ref, be2_ref, rm2_ref, rv2_ref),
    ):
        y = lax.dot_general(h, w_ref[...], (((1,), (1,)), ((), ())),
                            preferred_element_type=jnp.float32)
        y = y + b_ref[...][None, :]
        scale = (g_ref[...] * lax.rsqrt(rv_ref[...] + 1e-5))[None, :]
        y = (y - rm_ref[...][None, :]) * scale + be_ref[...][None, :]
        h = jnp.maximum(y, 0.0)
    o = lax.dot_general(h, w3_ref[...], (((1,), (1,)), ((), ())),
                        preferred_element_type=jnp.float32)
    out_ref[...] = o + b3_ref[0]


def _mlp(x, w0p, b0, g0, be0, rm0, rv0, w1, b1, g1, be1, rm1, rv1,
         w2, b2, g2, be2, rm2, rv2, w3, b3, block_b=512):
    grid = (BATCH // block_b,)

    def full2(shape):
        return pl.BlockSpec(shape, lambda i: (0, 0))

    def full1(shape):
        return pl.BlockSpec(shape, lambda i: (0,))

    h0, h1, h2 = w0p.shape[0], w1.shape[0], w2.shape[0]
    in_specs = [
        pl.BlockSpec((block_b, XW), lambda i: (i, 0)),
        full2(w0p.shape), full1((h0,)), full1((h0,)), full1((h0,)),
        full1((h0,)), full1((h0,)),
        full2(w1.shape), full1((h1,)), full1((h1,)), full1((h1,)),
        full1((h1,)), full1((h1,)),
        full2(w2.shape), full1((h2,)), full1((h2,)), full1((h2,)),
        full1((h2,)), full1((h2,)),
        full2(w3.shape),
        pl.BlockSpec(memory_space=pltpu.MemorySpace.SMEM),
    ]
    return pl.pallas_call(
        _mlp_body,
        grid=grid,
        in_specs=in_specs,
        out_specs=pl.BlockSpec((block_b, 8), lambda i: (i, 0)),
        out_shape=jax.ShapeDtypeStruct((BATCH, 8), jnp.float32),
    )(x, w0p, b0, g0, be0, rm0, rv0, w1, b1, g1, be1, rm1, rv1,
      w2, b2, g2, be2, rm2, rv2, w3, b3)


def _pack_source(tables, x_cont):
    parts = []
    for t, nch in zip(tables, NCH):
        n, d = t.shape
        tp = jnp.pad(t, ((0, 0), (0, nch * L - d)))
        parts.append(tp.reshape(n * nch, L))
    parts.append(jnp.pad(x_cont, ((0, 0), (0, L - N_CONT))))
    return jnp.concatenate(parts, axis=0)


def _chunk_indices(x_emb):
    xi = x_emb.astype(jnp.int32)
    bases = []
    base = 0
    for n, nch in zip(EMB_ROWS, NCH):
        bases.append(base)
        base += n * nch
    cols = []
    for f in range(len(EMB_ROWS)):
        for j in range(NCH[f]):
            cols.append(bases[f] + xi[:, f] * NCH[f] + j)
    cols.append(base + jnp.arange(BATCH, dtype=jnp.int32))
    nw = plsc.get_sparse_core_info().num_cores * plsc.get_sparse_core_info().num_subcores
    return jnp.stack(cols, axis=1).reshape(nw, -1, IDX_MINOR)


def _pad_w0(w0):
    segs = []
    off = 0
    for d, nch in zip(EMB_DIMS, NCH):
        segs.append(jnp.pad(w0[:, off:off + d], ((0, 0), (0, nch * L - d))))
        off += d
    segs.append(jnp.pad(w0[:, off:off + N_CONT], ((0, 0), (0, L - N_CONT))))
    return jnp.concatenate(segs, axis=1)


def kernel(x_emb, x_cont, emb0, emb1, emb2, emb3, emb4, emb5, emb6,
           W0, b0, g0, be0, rm0, rv0,
           W1, b1, g1, be1, rm1, rv1,
           W2, b2, g2, be2, rm2, rv2,
           W3, b3):
    tables = (emb0, emb1, emb2, emb3, emb4, emb5, emb6)
    src = _pack_source(tables, x_cont)
    cidx = _chunk_indices(x_emb)
    x = _sc_gather(src, cidx).reshape(BATCH, XW)
    w0p = _pad_w0(W0)
    w3p = jnp.pad(W3, ((0, 7), (0, 0)))
    out = _mlp(x, w0p, b0, g0, be0, rm0, rv0, W1, b1, g1, be1, rm1, rv1,
               W2, b2, g2, be2, rm2, rv2, w3p, b3)
    return out[:, 0]


# speedup vs baseline: 3.8643x; 2.6617x over previous
import functools

import jax
import jax.numpy as jnp
from jax import lax
from jax.experimental import pallas as pl
from jax.experimental.pallas import tpu as pltpu
from jax.experimental.pallas import tpu_sc as plsc

BATCH = 16384
L = 16
EMB_ROWS = (1559, 16, 5, 3, 4, 10, 4)
EMB_DIMS = (50, 8, 3, 2, 3, 5, 3)
NCH = tuple(-(-d // L) for d in EMB_DIMS)
N_CONT = 15
CHUNKS = sum(NCH)
XW = CHUNKS * L
R_TBL = sum(n * c for n, c in zip(EMB_ROWS, NCH))
IDX_MINOR = 128


def _sc_gather(src, cidx):
    info = plsc.get_sparse_core_info()
    nw = info.num_cores * info.num_subcores
    nidx = BATCH * CHUNKS
    per_w = nidx // nw
    n_chunk = per_w // IDX_MINOR
    mesh = plsc.VectorSubcoreMesh(core_axis_name="c", subcore_axis_name="s")

    @functools.partial(
        pl.kernel,
        mesh=mesh,
        compiler_params=pltpu.CompilerParams(use_tc_tiling_on_sc=False),
        out_type=jax.ShapeDtypeStruct((nidx, L), jnp.float32),
        scratch_types=[
            pltpu.VMEM((n_chunk, IDX_MINOR), jnp.int32),
            pltpu.VMEM((per_w, L), jnp.float32),
            pltpu.VMEM_SHARED((R_TBL, L), jnp.float32),
            pltpu.SemaphoreType.DMA,
        ],
    )
    def k(src_hbm, cidx_hbm, out_hbm, idx_v, rows_v, spt, sem):
        sub = lax.axis_index("s")
        wid = sub * info.num_cores + lax.axis_index("c")

        @pl.when(sub == 0)
        def _stage():
            pltpu.sync_copy(src_hbm, spt)

        pltpu.sync_copy(cidx_hbm.at[wid], idx_v)
        plsc.subcore_barrier()

        def body(j, carry):
            pltpu.async_copy(
                spt.at[idx_v.at[j]],
                rows_v.at[pl.ds(j * IDX_MINOR, IDX_MINOR)],
                sem,
            )
            return carry

        lax.fori_loop(0, n_chunk, body, 0)
        pltpu.make_async_copy(
            out_hbm.at[pl.ds(wid * per_w, per_w)], rows_v, sem
        ).wait()
        pltpu.sync_copy(rows_v, out_hbm.at[pl.ds(wid * per_w, per_w)])

    return k(src, cidx)


def _mlp_body(x_ref, xc_ref, w0_ref, wc_ref, b0_ref, g0_ref, be0_ref,
              rm0_ref, rv0_ref,
              w1_ref, b1_ref, g1_ref, be1_ref, rm1_ref, rv1_ref,
              w2_ref, b2_ref, g2_ref, be2_ref, rm2_ref, rv2_ref,
              w3_ref, b3_ref, out_ref):
    dn = (((1,), (1,)), ((), ()))
    h = lax.dot_general(x_ref[...], w0_ref[...], dn,
                        preferred_element_type=jnp.float32)
    h = h + lax.dot_general(xc_ref[...], wc_ref[...], dn,
                            preferred_element_type=jnp.float32)
    first = True
    for w_ref, b_ref, g_ref, be_ref, rm_ref, rv_ref in (
        (None, b0_ref, g0_ref, be0_ref, rm0_ref, rv0_ref),
        (w1_ref, b1_ref, g1_ref, be1_ref, rm1_ref, rv1_ref),
        (w2_ref, b2_ref, g2_ref, be2_ref, rm2_ref, rv2_ref),
    ):
        if not first:
            h = lax.dot_general(h, w_ref[...], dn,
                                preferred_element_type=jnp.float32)
        first = False
        y = h + b_ref[...][None, :]
        scale = (g_ref[...] * lax.rsqrt(rv_ref[...] + 1e-5))[None, :]
        y = (y - rm_ref[...][None, :]) * scale + be_ref[...][None, :]
        h = jnp.maximum(y, 0.0)
    o = lax.dot_general(h, w3_ref[...], dn, preferred_element_type=jnp.float32)
    out_ref[...] = o + b3_ref[0]


def _mlp(x, xc, w0e, w0c, b0, g0, be0, rm0, rv0, w1, b1, g1, be1, rm1, rv1,
         w2, b2, g2, be2, rm2, rv2, w3, b3, block_b=512):
    grid = (BATCH // block_b,)

    def full2(shape):
        return pl.BlockSpec(shape, lambda i: (0, 0))

    def full1(shape):
        return pl.BlockSpec(shape, lambda i: (0,))

    h0, h1, h2 = w0e.shape[0], w1.shape[0], w2.shape[0]
    in_specs = [
        pl.BlockSpec((block_b, XW), lambda i: (i, 0)),
        pl.BlockSpec((block_b, N_CONT), lambda i: (i, 0)),
        full2(w0e.shape), full2(w0c.shape), full1((h0,)), full1((h0,)),
        full1((h0,)), full1((h0,)), full1((h0,)),
        full2(w1.shape), full1((h1,)), full1((h1,)), full1((h1,)),
        full1((h1,)), full1((h1,)),
        full2(w2.shape), full1((h2,)), full1((h2,)), full1((h2,)),
        full1((h2,)), full1((h2,)),
        full2(w3.shape),
        pl.BlockSpec(memory_space=pltpu.MemorySpace.SMEM),
    ]
    return pl.pallas_call(
        _mlp_body,
        grid=grid,
        in_specs=in_specs,
        out_specs=pl.BlockSpec((block_b, 8), lambda i: (i, 0)),
        out_shape=jax.ShapeDtypeStruct((BATCH, 8), jnp.float32),
    )(x, xc, w0e, w0c, b0, g0, be0, rm0, rv0, w1, b1, g1, be1, rm1, rv1,
      w2, b2, g2, be2, rm2, rv2, w3, b3)


def _pack_source(tables):
    parts = []
    for t, nch in zip(tables, NCH):
        n, d = t.shape
        tp = jnp.pad(t, ((0, 0), (0, nch * L - d)))
        parts.append(tp.reshape(n * nch, L))
    return jnp.concatenate(parts, axis=0)


def _chunk_indices(x_emb):
    xi = x_emb.astype(jnp.int32)
    bases = []
    base = 0
    for n, nch in zip(EMB_ROWS, NCH):
        bases.append(base)
        base += n * nch
    cols = []
    for f in range(len(EMB_ROWS)):
        for j in range(NCH[f]):
            cols.append(bases[f] + xi[:, f] * NCH[f] + j)
    nw = plsc.get_sparse_core_info()
    nw = nw.num_cores * nw.num_subcores
    return jnp.stack(cols, axis=1).reshape(nw, -1, IDX_MINOR)


def _pad_w0e(w0):
    segs = []
    off = 0
    for d, nch in zip(EMB_DIMS, NCH):
        segs.append(jnp.pad(w0[:, off:off + d], ((0, 0), (0, nch * L - d))))
        off += d
    return jnp.concatenate(segs, axis=1)


def kernel(x_emb, x_cont, emb0, emb1, emb2, emb3, emb4, emb5, emb6,
           W0, b0, g0, be0, rm0, rv0,
           W1, b1, g1, be1, rm1, rv1,
           W2, b2, g2, be2, rm2, rv2,
           W3, b3):
    tables = (emb0, emb1, emb2, emb3, emb4, emb5, emb6)
    src = _pack_source(tables)
    cidx = _chunk_indices(x_emb)
    x = _sc_gather(src, cidx).reshape(BATCH, XW)
    w0e = _pad_w0e(W0)
    w0c = W0[:, sum(EMB_DIMS):]
    w3p = jnp.pad(W3, ((0, 7), (0, 0)))
    out = _mlp(x, x_cont, w0e, w0c, b0, g0, be0, rm0, rv0,
               W1, b1, g1, be1, rm1, rv1,
               W2, b2, g2, be2, rm2, rv2, w3p, b3)
    return out[:, 0]


# speedup vs baseline: 4.0370x; 1.0447x over previous
import functools

import jax
import jax.numpy as jnp
from jax import lax
from jax.experimental import pallas as pl
from jax.experimental.pallas import tpu as pltpu
from jax.experimental.pallas import tpu_sc as plsc

BATCH = 16384
L = 16
EMB_ROWS = (1559, 16, 5, 3, 4, 10, 4)
EMB_DIMS = (50, 8, 3, 2, 3, 5, 3)
NCH = tuple(-(-d // L) for d in EMB_DIMS)
N_CONT = 15
CHUNKS = sum(NCH)
XW = CHUNKS * L
R_TBL = sum(n * c for n, c in zip(EMB_ROWS, NCH))
IDX_MINOR = 128


def _sc_gather(src, cidx):
    info = plsc.get_sparse_core_info()
    nw = info.num_cores * info.num_subcores
    nidx = BATCH * CHUNKS
    per_w = nidx // nw
    n_chunk = per_w // IDX_MINOR
    mesh = plsc.VectorSubcoreMesh(core_axis_name="c", subcore_axis_name="s")

    @functools.partial(
        pl.kernel,
        mesh=mesh,
        compiler_params=pltpu.CompilerParams(use_tc_tiling_on_sc=False),
        out_type=jax.ShapeDtypeStruct((nidx, L), jnp.float32),
        scratch_types=[
            pltpu.VMEM((n_chunk, IDX_MINOR), jnp.int32),
            pltpu.VMEM((per_w, L), jnp.float32),
            pltpu.VMEM_SHARED((R_TBL, L), jnp.float32),
            pltpu.SemaphoreType.DMA,
        ],
    )
    def k(src_hbm, cidx_hbm, out_hbm, idx_v, rows_v, spt, sem):
        sub = lax.axis_index("s")
        wid = sub * info.num_cores + lax.axis_index("c")

        @pl.when(sub == 0)
        def _stage():
            pltpu.sync_copy(src_hbm, spt)

        pltpu.sync_copy(cidx_hbm.at[wid], idx_v)
        plsc.subcore_barrier()

        def body(j, carry):
            pltpu.async_copy(
                spt.at[idx_v.at[j]],
                rows_v.at[pl.ds(j * IDX_MINOR, IDX_MINOR)],
                sem,
            )
            return carry

        lax.fori_loop(0, n_chunk, body, 0)
        pltpu.make_async_copy(
            out_hbm.at[pl.ds(wid * per_w, per_w)], rows_v, sem
        ).wait()
        pltpu.sync_copy(rows_v, out_hbm.at[pl.ds(wid * per_w, per_w)])

    return k(src, cidx)


def _mlp_body(x_ref, xc_ref, w0_ref, wc_ref, b0_ref, g0_ref, be0_ref,
              rm0_ref, rv0_ref,
              w1_ref, b1_ref, g1_ref, be1_ref, rm1_ref, rv1_ref,
              w2_ref, b2_ref, g2_ref, be2_ref, rm2_ref, rv2_ref,
              w3_ref, b3_ref, out_ref):
    dn = (((1,), (1,)), ((), ()))
    h = lax.dot_general(x_ref[...], w0_ref[...], dn,
                        preferred_element_type=jnp.float32)
    h = h + lax.dot_general(xc_ref[...], wc_ref[...], dn,
                            preferred_element_type=jnp.float32)
    first = True
    for w_ref, b_ref, g_ref, be_ref, rm_ref, rv_ref in (
        (None, b0_ref, g0_ref, be0_ref, rm0_ref, rv0_ref),
        (w1_ref, b1_ref, g1_ref, be1_ref, rm1_ref, rv1_ref),
        (w2_ref, b2_ref, g2_ref, be2_ref, rm2_ref, rv2_ref),
    ):
        if not first:
            h = lax.dot_general(h, w_ref[...], dn,
                                preferred_element_type=jnp.float32)
        first = False
        y = h + b_ref[...][None, :]
        scale = (g_ref[...] * lax.rsqrt(rv_ref[...] + 1e-5))[None, :]
        y = (y - rm_ref[...][None, :]) * scale + be_ref[...][None, :]
        h = jnp.maximum(y, 0.0)
    o = lax.dot_general(h, w3_ref[...], dn, preferred_element_type=jnp.float32)
    out_ref[...] = o + b3_ref[0]


def _mlp(x, xc, w0e, w0c, b0, g0, be0, rm0, rv0, w1, b1, g1, be1, rm1, rv1,
         w2, b2, g2, be2, rm2, rv2, w3, b3, block_b=1024):
    grid = (BATCH // block_b,)

    def full2(shape):
        return pl.BlockSpec(shape, lambda i: (0, 0))

    def full1(shape):
        return pl.BlockSpec(shape, lambda i: (0,))

    h0, h1, h2 = w0e.shape[0], w1.shape[0], w2.shape[0]
    in_specs = [
        pl.BlockSpec((block_b, XW), lambda i: (i, 0)),
        pl.BlockSpec((block_b, N_CONT), lambda i: (i, 0)),
        full2(w0e.shape), full2(w0c.shape), full1((h0,)), full1((h0,)),
        full1((h0,)), full1((h0,)), full1((h0,)),
        full2(w1.shape), full1((h1,)), full1((h1,)), full1((h1,)),
        full1((h1,)), full1((h1,)),
        full2(w2.shape), full1((h2,)), full1((h2,)), full1((h2,)),
        full1((h2,)), full1((h2,)),
        full2(w3.shape),
        pl.BlockSpec(memory_space=pltpu.MemorySpace.SMEM),
    ]
    return pl.pallas_call(
        _mlp_body,
        grid=grid,
        in_specs=in_specs,
        out_specs=pl.BlockSpec((block_b, 8), lambda i: (i, 0)),
        out_shape=jax.ShapeDtypeStruct((BATCH, 8), jnp.float32),
    )(x, xc, w0e, w0c, b0, g0, be0, rm0, rv0, w1, b1, g1, be1, rm1, rv1,
      w2, b2, g2, be2, rm2, rv2, w3, b3)


def _pack_source(tables):
    parts = []
    for t, nch in zip(tables, NCH):
        n, d = t.shape
        tp = jnp.pad(t, ((0, 0), (0, nch * L - d)))
        parts.append(tp.reshape(n * nch, L))
    return jnp.concatenate(parts, axis=0)


def _chunk_indices(x_emb):
    xi = x_emb.astype(jnp.int32)
    bases = []
    base = 0
    for n, nch in zip(EMB_ROWS, NCH):
        bases.append(base)
        base += n * nch
    cols = []
    for f in range(len(EMB_ROWS)):
        for j in range(NCH[f]):
            cols.append(bases[f] + xi[:, f] * NCH[f] + j)
    nw = plsc.get_sparse_core_info()
    nw = nw.num_cores * nw.num_subcores
    return jnp.stack(cols, axis=1).reshape(nw, -1, IDX_MINOR)


def _pad_w0e(w0):
    segs = []
    off = 0
    for d, nch in zip(EMB_DIMS, NCH):
        segs.append(jnp.pad(w0[:, off:off + d], ((0, 0), (0, nch * L - d))))
        off += d
    return jnp.concatenate(segs, axis=1)


def kernel(x_emb, x_cont, emb0, emb1, emb2, emb3, emb4, emb5, emb6,
           W0, b0, g0, be0, rm0, rv0,
           W1, b1, g1, be1, rm1, rv1,
           W2, b2, g2, be2, rm2, rv2,
           W3, b3):
    tables = (emb0, emb1, emb2, emb3, emb4, emb5, emb6)
    src = _pack_source(tables)
    cidx = _chunk_indices(x_emb)
    x = _sc_gather(src, cidx).reshape(BATCH, XW)
    w0e = _pad_w0e(W0)
    w0c = W0[:, sum(EMB_DIMS):]
    w3p = jnp.pad(W3, ((0, 7), (0, 0)))
    out = _mlp(x, x_cont, w0e, w0c, b0, g0, be0, rm0, rv0,
               W1, b1, g1, be1, rm1, rv1,
               W2, b2, g2, be2, rm2, rv2, w3p, b3)
    return out[:, 0]
